# baseline (speedup 1.0000x reference)
import numpy as np
import jax
import jax.numpy as jnp
from jax.experimental import pallas as pl
from jax.experimental.pallas import tpu as pltpu

ROWS = 32
N = 1_000_000
NPAD = 1 << 20
SUB = 1024
LANE = 1024
SCH = 8
WCH = 8
TEMP = np.float32(0.7 + (1.5 - 0.7) * (1.0 - 0.5))
TOPP = np.float32(0.85 + (0.98 - 0.85) * (1.0 - 0.5))
TINY = np.float32(np.finfo(np.float32).tiny)
NEGINF = np.float32(-np.inf)

_ROTS = (13, 15, 26, 6, 17, 29, 16, 24)


def _threefry_bits(flat_u32):
    k0 = jnp.uint32(0)
    k1 = jnp.uint32(42)
    ks2 = jnp.uint32(0x1BD11BDA) ^ k0 ^ k1
    ks = (k0, k1, ks2)
    x0 = jnp.full_like(flat_u32, k0)
    x1 = flat_u32 + k1
    for r in range(20):
        x0 = x0 + x1
        d = _ROTS[r % 8]
        x1 = (x1 << jnp.uint32(d)) | (x1 >> jnp.uint32(32 - d))
        x1 = x1 ^ x0
        if r % 4 == 3:
            j = r // 4 + 1
            x0 = x0 + ks[j % 3]
            x1 = x1 + ks[(j + 1) % 3] + jnp.uint32(j)
    return x0 ^ x1


def _gumbel_from_bits(bits):
    fb = (bits >> jnp.uint32(9)) | jnp.uint32(0x3F800000)
    f = jax.lax.bitcast_convert_type(fb, jnp.float32) - jnp.float32(1.0)
    u = f * (jnp.float32(1.0) - TINY) + TINY
    u = jnp.maximum(TINY, u)
    return -jnp.log(-jnp.log(u))


def _row_kernel(l_ref, out_ref, e_ref, s_ref):
    row = pl.program_id(0)
    l = l_ref[...]
    s = l / TEMP
    s_ref[...] = s
    m = jnp.max(s, axis=(-2, -1), keepdims=True)
    e = jnp.exp(s - m)
    e_ref[...] = e
    z = jnp.sum(e, axis=(-2, -1), keepdims=True)
    target = TOPP * z

    def search_it(_, c):
        lo, hi = c
        mid = jnp.float32(0.5) * (lo + hi)

        ev = e_ref[...]
        w = jnp.sum(jnp.where(ev > mid, ev, jnp.float32(0.0)),
                    axis=(-2, -1), keepdims=True)
        above = w > target
        return jnp.where(above, mid, lo), jnp.where(above, hi, mid)

    lo0 = jnp.zeros((1, 1, 1), jnp.float32)
    hi0 = jnp.ones((1, 1, 1), jnp.float32)
    tau, _ = jax.lax.fori_loop(0, 30, search_it, (lo0, hi0))

    base = (row * N).astype(jnp.uint32)

    def samp_it(c, carry):
        best_v, best_i = carry
        shape = (1, SCH, LANE)
        col = ((jax.lax.broadcasted_iota(jnp.int32, shape, 1) + c * SCH) * LANE
               + jax.lax.broadcasted_iota(jnp.int32, shape, 2))
        flat = base + col.astype(jnp.uint32)
        g = _gumbel_from_bits(_threefry_bits(flat))
        sc = s_ref[:, pl.ds(c * SCH, SCH), :]
        ev = e_ref[:, pl.ds(c * SCH, SCH), :]
        vals = jnp.where(ev > tau, sc + g, NEGINF)
        cmax = jnp.max(vals, axis=(-2, -1), keepdims=True)
        cidx = jnp.min(jnp.where(vals == cmax, col, jnp.int32(2 ** 30)),
                       axis=(-2, -1), keepdims=True)
        better = cmax > best_v
        return (jnp.where(better, cmax, best_v),
                jnp.where(better, cidx, best_i))

    bv0 = jnp.full((1, 1, 1), NEGINF, jnp.float32)
    bi0 = jnp.zeros((1, 1, 1), jnp.int32)
    _, best_i = jax.lax.fori_loop(0, SUB // SCH, samp_it, (bv0, bi0))
    out_ref[...] = jnp.broadcast_to(best_i, (1, 1, 128)).astype(jnp.float32)


@jax.jit
def kernel(logits, x_context, W_srf, b_srf):
    del x_context, W_srf, b_srf
    lp = jnp.pad(logits, ((0, 0), (0, NPAD - N)), constant_values=NEGINF)
    lp = lp.reshape(ROWS, SUB, LANE)
    out = pl.pallas_call(
        _row_kernel,
        grid=(ROWS,),
        in_specs=[pl.BlockSpec((1, SUB, LANE), lambda r: (r, 0, 0))],
        out_specs=pl.BlockSpec((1, 1, 128), lambda r: (r, 0, 0)),
        out_shape=jax.ShapeDtypeStruct((ROWS, 1, 128), jnp.float32),
        scratch_shapes=[
            pltpu.VMEM((1, SUB, LANE), jnp.float32),
            pltpu.VMEM((1, SUB, LANE), jnp.float32),
        ],
        compiler_params=pltpu.CompilerParams(
            dimension_semantics=("arbitrary",),
        ),
    )(lp)
    return out[:, 0, :1].astype(jnp.int32)


# speedup vs baseline: 1.3645x; 1.3645x over previous
import numpy as np
import jax
import jax.numpy as jnp
from jax.experimental import pallas as pl
from jax.experimental.pallas import tpu as pltpu

ROWS = 32
N = 1_000_000
NPAD = 1 << 20
SUB = 1024
LANE = 1024
SCH = 128
TEMP = np.float32(0.7 + (1.5 - 0.7) * (1.0 - 0.5))
TOPP = np.float32(0.85 + (0.98 - 0.85) * (1.0 - 0.5))
TINY = np.float32(np.finfo(np.float32).tiny)
NEGINF = np.float32(-np.inf)

_ROTS = (13, 15, 26, 6, 17, 29, 16, 24)


def _threefry_bits(flat_u32):
    ks = (0, 42, 0x1BD11BDA ^ 0 ^ 42)
    x1 = flat_u32 + jnp.uint32(42)
    x0 = None
    for r in range(20):
        x0 = x1 if r == 0 else x0 + x1
        d = _ROTS[r % 8]
        x1 = (x1 << jnp.uint32(d)) | (x1 >> jnp.uint32(32 - d))
        x1 = x1 ^ x0
        if r % 4 == 3:
            j = r // 4 + 1
            if ks[j % 3]:
                x0 = x0 + jnp.uint32(ks[j % 3])
            x1 = x1 + jnp.uint32((ks[(j + 1) % 3] + j) & 0xFFFFFFFF)
    return x0 ^ x1


def _gumbel_from_bits(bits):
    fb = (bits >> jnp.uint32(9)) | jnp.uint32(0x3F800000)
    f = jax.lax.bitcast_convert_type(fb, jnp.float32) - jnp.float32(1.0)
    u = f * (jnp.float32(1.0) - TINY) + TINY
    u = jnp.maximum(TINY, u)
    return -jnp.log(-jnp.log(u))


def _row_kernel(l_ref, out_ref, e_ref, s_ref):
    row = pl.program_id(0)
    l = l_ref[...]
    s = l / TEMP
    s_ref[...] = s
    m = jnp.max(s, axis=(-2, -1), keepdims=True)
    e = jnp.exp(s - m)
    e_ref[...] = e
    z = jnp.sum(e, axis=(-2, -1), keepdims=True)
    target = TOPP * z

    def search_it(_, c):
        lo, hi = c
        mid = jnp.float32(0.5) * (lo + hi)

        ev = e_ref[...]
        w = jnp.sum(jnp.where(ev > mid, ev, jnp.float32(0.0)),
                    axis=(-2, -1), keepdims=True)
        above = w > target
        return jnp.where(above, mid, lo), jnp.where(above, hi, mid)

    lo0 = jnp.zeros((1, 1, 1), jnp.float32)
    hi0 = jnp.ones((1, 1, 1), jnp.float32)
    tau, _ = jax.lax.fori_loop(0, 26, search_it, (lo0, hi0))

    base = (row * N).astype(jnp.uint32)

    def samp_it(c, carry):
        best_v, best_i = carry
        shape = (1, SCH, LANE)
        col = ((jax.lax.broadcasted_iota(jnp.int32, shape, 1) + c * SCH) * LANE
               + jax.lax.broadcasted_iota(jnp.int32, shape, 2))
        flat = base + col.astype(jnp.uint32)
        g = _gumbel_from_bits(_threefry_bits(flat))
        sc = s_ref[:, pl.ds(c * SCH, SCH), :]
        ev = e_ref[:, pl.ds(c * SCH, SCH), :]
        vals = jnp.where(ev > tau, sc + g, NEGINF)
        cmax = jnp.max(vals, axis=(-2, -1), keepdims=True)
        cidx = jnp.min(jnp.where(vals == cmax, col, jnp.int32(2 ** 30)),
                       axis=(-2, -1), keepdims=True)
        better = cmax > best_v
        return (jnp.where(better, cmax, best_v),
                jnp.where(better, cidx, best_i))

    bv0 = jnp.full((1, 1, 1), NEGINF, jnp.float32)
    bi0 = jnp.zeros((1, 1, 1), jnp.int32)
    _, best_i = jax.lax.fori_loop(0, SUB // SCH, samp_it, (bv0, bi0))
    out_ref[...] = jnp.broadcast_to(best_i, (1, 1, 128)).astype(jnp.float32)


@jax.jit
def kernel(logits, x_context, W_srf, b_srf):
    del x_context, W_srf, b_srf
    lp = jnp.pad(logits, ((0, 0), (0, NPAD - N)), constant_values=NEGINF)
    lp = lp.reshape(ROWS, SUB, LANE)
    out = pl.pallas_call(
        _row_kernel,
        grid=(ROWS,),
        in_specs=[pl.BlockSpec((1, SUB, LANE), lambda r: (r, 0, 0))],
        out_specs=pl.BlockSpec((1, 1, 128), lambda r: (r, 0, 0)),
        out_shape=jax.ShapeDtypeStruct((ROWS, 1, 128), jnp.float32),
        scratch_shapes=[
            pltpu.VMEM((1, SUB, LANE), jnp.float32),
            pltpu.VMEM((1, SUB, LANE), jnp.float32),
        ],
        compiler_params=pltpu.CompilerParams(
            dimension_semantics=("arbitrary",),
        ),
    )(lp)
    return out[:, 0, :1].astype(jnp.int32)
